# baseline (speedup 1.0000x reference)
import jax
import jax.numpy as jnp
from jax.experimental import pallas as pl
from jax.experimental.pallas import tpu as pltpu

_G = 32


def _finer_kernel(idx_ref, coarse_ref, mask_ref, *rest):
    bank_refs = rest[:_G]
    out_ref = rest[_G]
    BS = bank_refs[0].shape[2]
    nbase = pl.program_id(1) * _G
    for j in range(_G):
        c = coarse_ref[0, pl.ds(nbase + j, 1), :]
        bk = bank_refs[j][0, 0]
        m = mask_ref[0, nbase + j]
        out_ref[0, j * BS:(j + 1) * BS] = (c - bk) * m


def _bank_spec(j, BS, D):
    return pl.BlockSpec(
        (1, 1, BS, D), lambda b, g, idx, j=j: (b, idx[b, g * _G + j], 0, 0))


def kernel(coarse_token_states, coarse_token_mask, fine_token_mask, bank, indice_table):
    B, NB, D = coarse_token_states.shape
    BS = bank.shape[2]
    mask4 = fine_token_mask.reshape(B, NB, BS, 1)

    out = pl.pallas_call(
        _finer_kernel,
        grid_spec=pltpu.PrefetchScalarGridSpec(
            num_scalar_prefetch=1,
            grid=(B, NB // _G),
            in_specs=[
                pl.BlockSpec((1, NB, D), lambda b, g, idx: (b, 0, 0)),
                pl.BlockSpec((1, NB, BS, 1), lambda b, g, idx: (b, 0, 0, 0)),
            ] + [_bank_spec(j, BS, D) for j in range(_G)],
            out_specs=pl.BlockSpec((1, _G * BS, D), lambda b, g, idx: (b, g, 0)),
        ),
        out_shape=jax.ShapeDtypeStruct((B, NB * BS, D), coarse_token_states.dtype),
    )(indice_table, coarse_token_states, mask4, *([bank] * _G))
    return out
